# baseline (speedup 1.0000x reference)
import functools

import numpy as np
import jax
import jax.numpy as jnp
from jax import lax
from jax.experimental import pallas as pl
from jax.experimental.pallas import tpu as pltpu
from jax.experimental.pallas import tpu_sc as plsc

_N = 10000
_E = 320000
_F = 128
_G = 64
_NSPL = 8
_NC = 2
_NS = 16
_NW = _NC * _NS
_CHUNK = 64
_EDGES_PER_TILE = 10240
_EP = _NW * _EDGES_PER_TILE
_CHUNKS_PER_TILE = _EDGES_PER_TILE // _CHUNK
_STAGE_CHUNKS = 80
_IDX_ROWS = _EP // _CHUNK
_ACC_ROWS = 10240
_ZROWS = _ACC_ROWS // _NS
_B = 1000
_NBLK = _N // _B

_GRIDV = [float(v) for v in
          (np.arange(-3, 9, dtype=np.float32) * np.float32(2.0 / 5)
           - np.float32(1.0))]


def _sc_segment_sum(h, src2d, dst2d, zblk):
    mesh = plsc.VectorSubcoreMesh(core_axis_name="c", subcore_axis_name="s")

    @functools.partial(
        pl.kernel,
        out_type=jax.ShapeDtypeStruct((_NC, _ACC_ROWS, _F), jnp.float32),
        mesh=mesh,
        scratch_types=[
            pltpu.VMEM((_STAGE_CHUNKS, _CHUNK), jnp.int32),
            pltpu.VMEM((_STAGE_CHUNKS, _CHUNK), jnp.int32),
            pltpu.VMEM((_CHUNK, _F), jnp.float32),
            pltpu.VMEM((_CHUNK, _F), jnp.float32),
            pltpu.VMEM((_CHUNK, _F), jnp.float32),
            pltpu.VMEM_SHARED((_ACC_ROWS, _F), jnp.float32),
            pltpu.SemaphoreType.DMA,
            pltpu.SemaphoreType.DMA,
            pltpu.SemaphoreType.DMA,
        ],
    )
    def seg_kernel(h_hbm, src_hbm, dst_hbm, z_hbm, out_hbm,
                   sidx, didx, rows0, rows1, rows2, acc,
                   gsem0, gsem1, gsem2):
        c = lax.axis_index("c")
        s = lax.axis_index("s")
        rows = [rows0, rows1, rows2]
        gsem = [gsem0, gsem1, gsem2]
        pltpu.sync_copy(z_hbm, acc.at[pl.ds(s * _ZROWS, _ZROWS)])
        row0 = (c * _NS + s) * _CHUNKS_PER_TILE
        plsc.subcore_barrier()

        def wait_g(b):
            pltpu.make_async_copy(h_hbm.at[sidx.at[0]], rows[b],
                                  gsem[b]).wait()

        for t in range(_CHUNKS_PER_TILE // _STAGE_CHUNKS):
            pltpu.sync_copy(
                src_hbm.at[pl.ds(row0 + t * _STAGE_CHUNKS, _STAGE_CHUNKS)],
                sidx)
            pltpu.sync_copy(
                dst_hbm.at[pl.ds(row0 + t * _STAGE_CHUNKS, _STAGE_CHUNKS)],
                didx)
            pltpu.async_copy(h_hbm.at[sidx.at[0]], rows[0], gsem[0])
            pltpu.async_copy(h_hbm.at[sidx.at[1]], rows[1], gsem[1])

            @pl.loop(0, (_STAGE_CHUNKS - 2) // 3)
            def _(k):
                for b in range(3):
                    cidx = k * 3 + b
                    wait_g(b)
                    pltpu.sync_copy(rows[b], acc.at[didx.at[cidx]],
                                    add=True)
                    nb = (b + 2) % 3
                    pltpu.async_copy(h_hbm.at[sidx.at[cidx + 2]], rows[nb],
                                     gsem[nb])

            for cidx in (_STAGE_CHUNKS - 2, _STAGE_CHUNKS - 1):
                b = cidx % 3
                wait_g(b)
                pltpu.sync_copy(rows[b], acc.at[didx.at[cidx]], add=True)

        plsc.subcore_barrier()
        pltpu.sync_copy(acc.at[pl.ds(s * _ZROWS, _ZROWS)],
                        out_hbm.at[c, pl.ds(s * _ZROWS, _ZROWS)])

    return seg_kernel(h, src2d, dst2d, zblk)


def _kan(x, w_ref):
    g = _GRIDV
    sig = 1.0 / (1.0 + jnp.exp(-x))
    t = (x - g[0]) * (1.0 / (g[1] - g[0]))
    i = jnp.floor(t)
    u = t - i
    u2 = u * u
    u3 = u2 * u
    v = 1.0 - u
    s6 = 1.0 / 6.0
    w3 = u3 * s6
    w2 = s6 + 0.5 * (u + u2 - u3)
    w1 = 2.0 / 3.0 - u2 + 0.5 * u3
    w0 = v * v * v * s6
    e = [i == float(k) for k in range(11)]
    zero = jnp.zeros_like(x)
    cols = []
    for j in range(_NSPL):
        a = jnp.where(e[j], w3, zero)
        a = jnp.where(e[j + 1], w2, a)
        a = jnp.where(e[j + 2], w1, a)
        a = jnp.where(e[j + 3], w0, a)
        cols.append(a)
    feats = jnp.concatenate([x * sig] + cols, axis=1)
    return lax.dot_general(feats, w_ref[...], (((1,), (0,)), ((), ())),
                           preferred_element_type=jnp.float32)


def _layer_compute(eps_ref, h_ref, a_ref, w1_ref, w2_ref):
    eps = eps_ref[0, 0]
    h2 = (1.0 + eps) * h_ref[...] + a_ref[0] + a_ref[1]
    return _kan(_kan(h2, w1_ref), w2_ref)


def _mid_body(eps_ref, h_ref, a_ref, w1_ref, w2_ref, o_ref):
    t = _layer_compute(eps_ref, h_ref, a_ref, w1_ref, w2_ref)
    o_ref[...] = jnp.where(t >= 0, t, 0.01 * t)


def _last_body(eps_ref, h_ref, a_ref, w1_ref, w2_ref,
               b_ref, pool_ref, cnt_ref):
    i = pl.program_id(0)

    @pl.when(i == 0)
    def _():
        pool_ref[...] = jnp.zeros_like(pool_ref)
        cnt_ref[...] = jnp.zeros_like(cnt_ref)

    t = _layer_compute(eps_ref, h_ref, a_ref, w1_ref, w2_ref)
    bvec = b_ref[0, 0, :]
    onehot = (bvec[:, None]
              == lax.broadcasted_iota(jnp.int32, (_B, _G), 1)
              ).astype(jnp.float32)
    pool_ref[...] += lax.dot_general(
        onehot, t, (((0,), (0,)), ((), ())),
        preferred_element_type=jnp.float32)
    cnt_ref[...] += lax.dot_general(
        onehot, jnp.ones((_B, _F), jnp.float32), (((0,), (0,)), ((), ())),
        preferred_element_type=jnp.float32)

    @pl.when(i == _NBLK - 1)
    def _():
        pool_ref[...] = pool_ref[...] / jnp.maximum(cnt_ref[...], 1.0)


_EPS_SPEC = pl.BlockSpec((1, 1), lambda i: (0, 0))
_H_SPEC = pl.BlockSpec((_B, _F), lambda i: (i, 0))
_A_SPEC = pl.BlockSpec((_NC, _B, _F), lambda i: (0, i, 0))
_W_SPEC = pl.BlockSpec(((_NSPL + 1) * _F, _F), lambda i: (0, 0))


def _tc_mid(eps11, h, agg, w1, w2):
    return pl.pallas_call(
        _mid_body,
        grid=(_NBLK,),
        in_specs=[_EPS_SPEC, _H_SPEC, _A_SPEC, _W_SPEC, _W_SPEC],
        out_specs=_H_SPEC,
        out_shape=jax.ShapeDtypeStruct((_N, _F), jnp.float32),
    )(eps11, h, agg, w1, w2)


def _tc_last(eps11, h, agg, w1, w2, batch3):
    return pl.pallas_call(
        _last_body,
        grid=(_NBLK,),
        in_specs=[_EPS_SPEC, _H_SPEC, _A_SPEC, _W_SPEC, _W_SPEC,
                  pl.BlockSpec((1, 1, _B), lambda i: (i, 0, 0))],
        out_specs=pl.BlockSpec((_G, _F), lambda i: (0, 0)),
        out_shape=jax.ShapeDtypeStruct((_G, _F), jnp.float32),
        scratch_shapes=[pltpu.VMEM((_G, _F), jnp.float32)],
    )(eps11, h, agg, w1, w2, batch3)


def kernel(x, edge_index, batch, params):
    src = edge_index[0].astype(jnp.int32)
    dst = edge_index[1].astype(jnp.int32)
    pad = _EP - _E
    pad_ar = np.arange(pad)
    pad_src = jnp.asarray((pad_ar * 997) % _N, jnp.int32)
    pad_dst = jnp.asarray(_N + pad_ar % (_ACC_ROWS - _N), jnp.int32)
    src2d = jnp.concatenate([src, pad_src]).reshape(_IDX_ROWS, _CHUNK)
    dst2d = jnp.concatenate([dst, pad_dst]).reshape(_IDX_ROWS, _CHUNK)
    zblk = jnp.zeros((_ZROWS, _F), jnp.float32)
    batch3 = batch.astype(jnp.int32).reshape(_NBLK, 1, _B)

    def wcat(p):
        ws = jnp.transpose(p[1] * p[2][..., None], (2, 1, 0))
        return jnp.concatenate([p[0].T, ws.reshape(_NSPL * _F, _F)], axis=0)

    layer_args = []
    for (eps, p0, p1) in params:
        layer_args.append((
            jnp.reshape(eps, (1, 1)).astype(jnp.float32),
            wcat(p0),
            wcat(p1),
        ))

    h = x
    pooled = None
    for li in range(3):
        eps11, w1, w2 = layer_args[li]
        agg = _sc_segment_sum(h, src2d, dst2d, zblk)
        if li < 2:
            h = _tc_mid(eps11, h, agg, w1, w2)
        else:
            pooled = _tc_last(eps11, h, agg, w1, w2, batch3)
    return (pooled, 0)


# speedup vs baseline: 1.1466x; 1.1466x over previous
import functools

import numpy as np
import jax
import jax.numpy as jnp
from jax import lax
from jax.experimental import pallas as pl
from jax.experimental.pallas import tpu as pltpu
from jax.experimental.pallas import tpu_sc as plsc

_N = 10000
_E = 320000
_F = 128
_G = 64
_NSPL = 8
_NC = 2
_NS = 16
_NW = _NC * _NS
_CHUNK = 128
_EDGES_PER_TILE = 10240
_EP = _NW * _EDGES_PER_TILE
_CHUNKS_PER_TILE = _EDGES_PER_TILE // _CHUNK
_STAGE_CHUNKS = 40
_IDX_ROWS = _EP // _CHUNK
_ACC_ROWS = 10240
_ZROWS = _ACC_ROWS // _NS
_B = 1000
_NBLK = _N // _B

_GRIDV = [float(v) for v in
          (np.arange(-3, 9, dtype=np.float32) * np.float32(2.0 / 5)
           - np.float32(1.0))]


def _sc_segment_sum(h, src2d, dst2d, zblk):
    mesh = plsc.VectorSubcoreMesh(core_axis_name="c", subcore_axis_name="s")

    @functools.partial(
        pl.kernel,
        out_type=jax.ShapeDtypeStruct((_NC, _ACC_ROWS, _F), jnp.float32),
        mesh=mesh,
        scratch_types=[
            pltpu.VMEM((_STAGE_CHUNKS, _CHUNK), jnp.int32),
            pltpu.VMEM((_STAGE_CHUNKS, _CHUNK), jnp.int32),
            pltpu.VMEM((_CHUNK, _F), jnp.float32),
            pltpu.VMEM((_CHUNK, _F), jnp.float32),
            pltpu.VMEM_SHARED((_ACC_ROWS, _F), jnp.float32),
            pltpu.SemaphoreType.DMA,
            pltpu.SemaphoreType.DMA,
        ],
    )
    def seg_kernel(h_hbm, src_hbm, dst_hbm, z_hbm, out_hbm,
                   sidx, didx, rows0, rows1, acc,
                   gsem0, gsem1):
        c = lax.axis_index("c")
        s = lax.axis_index("s")
        rows = [rows0, rows1]
        gsem = [gsem0, gsem1]
        pltpu.sync_copy(z_hbm, acc.at[pl.ds(s * _ZROWS, _ZROWS)])
        row0 = (c * _NS + s) * _CHUNKS_PER_TILE
        plsc.subcore_barrier()

        def wait_g(b):
            pltpu.make_async_copy(h_hbm.at[sidx.at[0]], rows[b],
                                  gsem[b]).wait()

        for t in range(_CHUNKS_PER_TILE // _STAGE_CHUNKS):
            pltpu.sync_copy(
                src_hbm.at[pl.ds(row0 + t * _STAGE_CHUNKS, _STAGE_CHUNKS)],
                sidx)
            pltpu.sync_copy(
                dst_hbm.at[pl.ds(row0 + t * _STAGE_CHUNKS, _STAGE_CHUNKS)],
                didx)
            pltpu.async_copy(h_hbm.at[sidx.at[0]], rows[0], gsem[0])

            @pl.loop(0, _STAGE_CHUNKS // 2)
            def _(k):
                j = k * 2
                pltpu.async_copy(h_hbm.at[sidx.at[j + 1]], rows[1], gsem[1])
                wait_g(0)
                pltpu.sync_copy(rows[0], acc.at[didx.at[j]], add=True)

                @pl.when(j + 2 < _STAGE_CHUNKS)
                def _():
                    pltpu.async_copy(h_hbm.at[sidx.at[j + 2]], rows[0],
                                     gsem[0])

                wait_g(1)
                pltpu.sync_copy(rows[1], acc.at[didx.at[j + 1]], add=True)

        plsc.subcore_barrier()
        pltpu.sync_copy(acc.at[pl.ds(s * _ZROWS, _ZROWS)],
                        out_hbm.at[c, pl.ds(s * _ZROWS, _ZROWS)])

    return seg_kernel(h, src2d, dst2d, zblk)


def _kan(x, w_ref):
    g = _GRIDV
    sig = 1.0 / (1.0 + jnp.exp(-x))
    t = (x - g[0]) * (1.0 / (g[1] - g[0]))
    i = jnp.floor(t)
    u = t - i
    u2 = u * u
    u3 = u2 * u
    v = 1.0 - u
    s6 = 1.0 / 6.0
    w3 = u3 * s6
    w2 = s6 + 0.5 * (u + u2 - u3)
    w1 = 2.0 / 3.0 - u2 + 0.5 * u3
    w0 = v * v * v * s6
    e = [i == float(k) for k in range(11)]
    zero = jnp.zeros_like(x)
    cols = []
    for j in range(_NSPL):
        a = jnp.where(e[j], w3, zero)
        a = jnp.where(e[j + 1], w2, a)
        a = jnp.where(e[j + 2], w1, a)
        a = jnp.where(e[j + 3], w0, a)
        cols.append(a)
    feats = jnp.concatenate([x * sig] + cols, axis=1)
    return lax.dot_general(feats, w_ref[...], (((1,), (0,)), ((), ())),
                           preferred_element_type=jnp.float32)


def _layer_compute(eps_ref, h_ref, a_ref, w1_ref, w2_ref):
    eps = eps_ref[0, 0]
    h2 = (1.0 + eps) * h_ref[...] + a_ref[0] + a_ref[1]
    return _kan(_kan(h2, w1_ref), w2_ref)


def _mid_body(eps_ref, h_ref, a_ref, w1_ref, w2_ref, o_ref):
    t = _layer_compute(eps_ref, h_ref, a_ref, w1_ref, w2_ref)
    o_ref[...] = jnp.where(t >= 0, t, 0.01 * t)


def _last_body(eps_ref, h_ref, a_ref, w1_ref, w2_ref,
               b_ref, pool_ref, cnt_ref):
    i = pl.program_id(0)

    @pl.when(i == 0)
    def _():
        pool_ref[...] = jnp.zeros_like(pool_ref)
        cnt_ref[...] = jnp.zeros_like(cnt_ref)

    t = _layer_compute(eps_ref, h_ref, a_ref, w1_ref, w2_ref)
    bvec = b_ref[0, 0, :]
    onehot = (bvec[:, None]
              == lax.broadcasted_iota(jnp.int32, (_B, _G), 1)
              ).astype(jnp.float32)
    pool_ref[...] += lax.dot_general(
        onehot, t, (((0,), (0,)), ((), ())),
        preferred_element_type=jnp.float32)
    cnt_ref[...] += lax.dot_general(
        onehot, jnp.ones((_B, _F), jnp.float32), (((0,), (0,)), ((), ())),
        preferred_element_type=jnp.float32)

    @pl.when(i == _NBLK - 1)
    def _():
        pool_ref[...] = pool_ref[...] / jnp.maximum(cnt_ref[...], 1.0)


_EPS_SPEC = pl.BlockSpec((1, 1), lambda i: (0, 0))
_H_SPEC = pl.BlockSpec((_B, _F), lambda i: (i, 0))
_A_SPEC = pl.BlockSpec((_NC, _B, _F), lambda i: (0, i, 0))
_W_SPEC = pl.BlockSpec(((_NSPL + 1) * _F, _F), lambda i: (0, 0))


def _tc_mid(eps11, h, agg, w1, w2):
    return pl.pallas_call(
        _mid_body,
        grid=(_NBLK,),
        in_specs=[_EPS_SPEC, _H_SPEC, _A_SPEC, _W_SPEC, _W_SPEC],
        out_specs=_H_SPEC,
        out_shape=jax.ShapeDtypeStruct((_N, _F), jnp.float32),
    )(eps11, h, agg, w1, w2)


def _tc_last(eps11, h, agg, w1, w2, batch3):
    return pl.pallas_call(
        _last_body,
        grid=(_NBLK,),
        in_specs=[_EPS_SPEC, _H_SPEC, _A_SPEC, _W_SPEC, _W_SPEC,
                  pl.BlockSpec((1, 1, _B), lambda i: (i, 0, 0))],
        out_specs=pl.BlockSpec((_G, _F), lambda i: (0, 0)),
        out_shape=jax.ShapeDtypeStruct((_G, _F), jnp.float32),
        scratch_shapes=[pltpu.VMEM((_G, _F), jnp.float32)],
    )(eps11, h, agg, w1, w2, batch3)


def kernel(x, edge_index, batch, params):
    src = edge_index[0].astype(jnp.int32)
    dst = edge_index[1].astype(jnp.int32)
    pad = _EP - _E
    pad_ar = np.arange(pad)
    pad_src = jnp.asarray((pad_ar * 997) % _N, jnp.int32)
    pad_dst = jnp.asarray(_N + pad_ar % (_ACC_ROWS - _N), jnp.int32)
    src2d = jnp.concatenate([src, pad_src]).reshape(_IDX_ROWS, _CHUNK)
    dst2d = jnp.concatenate([dst, pad_dst]).reshape(_IDX_ROWS, _CHUNK)
    zblk = jnp.zeros((_ZROWS, _F), jnp.float32)
    batch3 = batch.astype(jnp.int32).reshape(_NBLK, 1, _B)

    def wcat(p):
        ws = jnp.transpose(p[1] * p[2][..., None], (2, 1, 0))
        return jnp.concatenate([p[0].T, ws.reshape(_NSPL * _F, _F)], axis=0)

    layer_args = []
    for (eps, p0, p1) in params:
        layer_args.append((
            jnp.reshape(eps, (1, 1)).astype(jnp.float32),
            wcat(p0),
            wcat(p1),
        ))

    h = x
    pooled = None
    for li in range(3):
        eps11, w1, w2 = layer_args[li]
        agg = _sc_segment_sum(h, src2d, dst2d, zblk)
        if li < 2:
            h = _tc_mid(eps11, h, agg, w1, w2)
        else:
            pooled = _tc_last(eps11, h, agg, w1, w2, batch3)
    return (pooled, 0)


# speedup vs baseline: 1.1590x; 1.0108x over previous
import functools

import numpy as np
import jax
import jax.numpy as jnp
from jax import lax
from jax.experimental import pallas as pl
from jax.experimental.pallas import tpu as pltpu
from jax.experimental.pallas import tpu_sc as plsc

_N = 10000
_E = 320000
_F = 128
_G = 64
_NSPL = 8
_NC = 2
_NS = 16
_NW = _NC * _NS
_CHUNK = 128
_EDGES_PER_TILE = 10240
_EP = _NW * _EDGES_PER_TILE
_CHUNKS_PER_TILE = _EDGES_PER_TILE // _CHUNK
_STAGE_CHUNKS = 40
_IDX_ROWS = _EP // _CHUNK
_ACC_ROWS = 10240
_ZROWS = _ACC_ROWS // _NS
_B = 2000
_NBLK = _N // _B

_GRIDV = [float(v) for v in
          (np.arange(-3, 9, dtype=np.float32) * np.float32(2.0 / 5)
           - np.float32(1.0))]


def _sc_segment_sum(h, src2d, dst2d, zblk):
    mesh = plsc.VectorSubcoreMesh(core_axis_name="c", subcore_axis_name="s")

    @functools.partial(
        pl.kernel,
        out_type=jax.ShapeDtypeStruct((_NC, _ACC_ROWS, _F), jnp.float32),
        mesh=mesh,
        scratch_types=[
            pltpu.VMEM((_STAGE_CHUNKS, _CHUNK), jnp.int32),
            pltpu.VMEM((_STAGE_CHUNKS, _CHUNK), jnp.int32),
            pltpu.VMEM((_CHUNK, _F), jnp.float32),
            pltpu.VMEM((_CHUNK, _F), jnp.float32),
            pltpu.VMEM_SHARED((_ACC_ROWS, _F), jnp.float32),
            pltpu.SemaphoreType.DMA,
            pltpu.SemaphoreType.DMA,
        ],
    )
    def seg_kernel(h_hbm, src_hbm, dst_hbm, z_hbm, out_hbm,
                   sidx, didx, rows0, rows1, acc,
                   gsem0, gsem1):
        c = lax.axis_index("c")
        s = lax.axis_index("s")
        rows = [rows0, rows1]
        gsem = [gsem0, gsem1]
        pltpu.sync_copy(z_hbm, acc.at[pl.ds(s * _ZROWS, _ZROWS)])
        row0 = (c * _NS + s) * _CHUNKS_PER_TILE
        plsc.subcore_barrier()

        def wait_g(b):
            pltpu.make_async_copy(h_hbm.at[sidx.at[0]], rows[b],
                                  gsem[b]).wait()

        for t in range(_CHUNKS_PER_TILE // _STAGE_CHUNKS):
            pltpu.sync_copy(
                src_hbm.at[pl.ds(row0 + t * _STAGE_CHUNKS, _STAGE_CHUNKS)],
                sidx)
            pltpu.sync_copy(
                dst_hbm.at[pl.ds(row0 + t * _STAGE_CHUNKS, _STAGE_CHUNKS)],
                didx)
            pltpu.async_copy(h_hbm.at[sidx.at[0]], rows[0], gsem[0])

            @pl.loop(0, _STAGE_CHUNKS // 2)
            def _(k):
                j = k * 2
                pltpu.async_copy(h_hbm.at[sidx.at[j + 1]], rows[1], gsem[1])
                wait_g(0)
                pltpu.sync_copy(rows[0], acc.at[didx.at[j]], add=True)

                @pl.when(j + 2 < _STAGE_CHUNKS)
                def _():
                    pltpu.async_copy(h_hbm.at[sidx.at[j + 2]], rows[0],
                                     gsem[0])

                wait_g(1)
                pltpu.sync_copy(rows[1], acc.at[didx.at[j + 1]], add=True)

        plsc.subcore_barrier()
        pltpu.sync_copy(acc.at[pl.ds(s * _ZROWS, _ZROWS)],
                        out_hbm.at[c, pl.ds(s * _ZROWS, _ZROWS)])

    return seg_kernel(h, src2d, dst2d, zblk)


def _kan(x, w_ref):
    g = _GRIDV
    sig = 1.0 / (1.0 + jnp.exp(-x))
    t = (x - g[0]) * (1.0 / (g[1] - g[0]))
    i = jnp.floor(t)
    u = t - i
    u2 = u * u
    u3 = u2 * u
    v = 1.0 - u
    s6 = 1.0 / 6.0
    w3 = u3 * s6
    w2 = s6 + 0.5 * (u + u2 - u3)
    w1 = 2.0 / 3.0 - u2 + 0.5 * u3
    w0 = v * v * v * s6
    e = [i == float(k) for k in range(11)]
    zero = jnp.zeros_like(x)
    cols = []
    for j in range(_NSPL):
        a = jnp.where(e[j], w3, zero)
        a = jnp.where(e[j + 1], w2, a)
        a = jnp.where(e[j + 2], w1, a)
        a = jnp.where(e[j + 3], w0, a)
        cols.append(a)
    feats = jnp.concatenate([x * sig] + cols, axis=1)
    return lax.dot_general(feats, w_ref[...], (((1,), (0,)), ((), ())),
                           preferred_element_type=jnp.float32)


def _layer_compute(eps_ref, h_ref, a_ref, w1_ref, w2_ref):
    eps = eps_ref[0, 0]
    h2 = (1.0 + eps) * h_ref[...] + a_ref[0] + a_ref[1]
    return _kan(_kan(h2, w1_ref), w2_ref)


def _mid_body(eps_ref, h_ref, a_ref, w1_ref, w2_ref, o_ref):
    t = _layer_compute(eps_ref, h_ref, a_ref, w1_ref, w2_ref)
    o_ref[...] = jnp.where(t >= 0, t, 0.01 * t)


def _last_body(eps_ref, h_ref, a_ref, w1_ref, w2_ref,
               b_ref, pool_ref, cnt_ref):
    i = pl.program_id(0)

    @pl.when(i == 0)
    def _():
        pool_ref[...] = jnp.zeros_like(pool_ref)
        cnt_ref[...] = jnp.zeros_like(cnt_ref)

    t = _layer_compute(eps_ref, h_ref, a_ref, w1_ref, w2_ref)
    bvec = b_ref[0, 0, :]
    onehot = (bvec[:, None]
              == lax.broadcasted_iota(jnp.int32, (_B, _G), 1)
              ).astype(jnp.float32)
    pool_ref[...] += lax.dot_general(
        onehot, t, (((0,), (0,)), ((), ())),
        preferred_element_type=jnp.float32)
    cnt_ref[...] += lax.dot_general(
        onehot, jnp.ones((_B, _F), jnp.float32), (((0,), (0,)), ((), ())),
        preferred_element_type=jnp.float32)

    @pl.when(i == _NBLK - 1)
    def _():
        pool_ref[...] = pool_ref[...] / jnp.maximum(cnt_ref[...], 1.0)


_EPS_SPEC = pl.BlockSpec((1, 1), lambda i: (0, 0))
_H_SPEC = pl.BlockSpec((_B, _F), lambda i: (i, 0))
_A_SPEC = pl.BlockSpec((_NC, _B, _F), lambda i: (0, i, 0))
_W_SPEC = pl.BlockSpec(((_NSPL + 1) * _F, _F), lambda i: (0, 0))


def _tc_mid(eps11, h, agg, w1, w2):
    return pl.pallas_call(
        _mid_body,
        grid=(_NBLK,),
        in_specs=[_EPS_SPEC, _H_SPEC, _A_SPEC, _W_SPEC, _W_SPEC],
        out_specs=_H_SPEC,
        out_shape=jax.ShapeDtypeStruct((_N, _F), jnp.float32),
    )(eps11, h, agg, w1, w2)


def _tc_last(eps11, h, agg, w1, w2, batch3):
    return pl.pallas_call(
        _last_body,
        grid=(_NBLK,),
        in_specs=[_EPS_SPEC, _H_SPEC, _A_SPEC, _W_SPEC, _W_SPEC,
                  pl.BlockSpec((1, 1, _B), lambda i: (i, 0, 0))],
        out_specs=pl.BlockSpec((_G, _F), lambda i: (0, 0)),
        out_shape=jax.ShapeDtypeStruct((_G, _F), jnp.float32),
        scratch_shapes=[pltpu.VMEM((_G, _F), jnp.float32)],
    )(eps11, h, agg, w1, w2, batch3)


def kernel(x, edge_index, batch, params):
    src = edge_index[0].astype(jnp.int32)
    dst = edge_index[1].astype(jnp.int32)
    pad = _EP - _E
    pad_ar = np.arange(pad)
    pad_src = jnp.asarray((pad_ar * 997) % _N, jnp.int32)
    pad_dst = jnp.asarray(_N + pad_ar % (_ACC_ROWS - _N), jnp.int32)
    src2d = jnp.concatenate([src, pad_src]).reshape(_IDX_ROWS, _CHUNK)
    dst2d = jnp.concatenate([dst, pad_dst]).reshape(_IDX_ROWS, _CHUNK)
    zblk = jnp.zeros((_ZROWS, _F), jnp.float32)
    batch3 = batch.astype(jnp.int32).reshape(_NBLK, 1, _B)

    def wcat(p):
        ws = jnp.transpose(p[1] * p[2][..., None], (2, 1, 0))
        return jnp.concatenate([p[0].T, ws.reshape(_NSPL * _F, _F)], axis=0)

    layer_args = []
    for (eps, p0, p1) in params:
        layer_args.append((
            jnp.reshape(eps, (1, 1)).astype(jnp.float32),
            wcat(p0),
            wcat(p1),
        ))

    h = x
    pooled = None
    for li in range(3):
        eps11, w1, w2 = layer_args[li]
        agg = _sc_segment_sum(h, src2d, dst2d, zblk)
        if li < 2:
            h = _tc_mid(eps11, h, agg, w1, w2)
        else:
            pooled = _tc_last(eps11, h, agg, w1, w2, batch3)
    return (pooled, 0)
